# Initial kernel scaffold; baseline (speedup 1.0000x reference)
import jax
import jax.numpy as jnp
from jax.experimental import pallas as pl


def kernel(d_coarse):
    raise NotImplementedError("write your pallas kernel here")


# speedup vs baseline: 26.1287x; 26.1287x over previous
import functools

import jax
import jax.numpy as jnp
from jax.experimental import pallas as pl
from jax.experimental.pallas import tpu as pltpu

_M = 2048
_BM = 128
_BA = 256


def _pool_kernel(x_ref, noise_ref, e_ref, mn_ref, mx_ref):
    i = pl.program_id(0)
    x = x_ref[...]
    y = jnp.maximum(x[:, :4096], x[:, 4096:])
    yt = y.T
    g = yt.reshape(2048, 2 * _BM)
    pt = jnp.maximum(g[:, :_BM], g[:, _BM:])
    bmin = jnp.min(pt)
    bmax = jnp.max(pt)

    @pl.when(i == 0)
    def _init():
        mn_ref[0, 0] = bmin
        mx_ref[0, 0] = bmax

    @pl.when(i > 0)
    def _acc():
        mn_ref[0, 0] = jnp.minimum(mn_ref[0, 0], bmin)
        mx_ref[0, 0] = jnp.maximum(mx_ref[0, 0], bmax)

    e_ref[...] = pt + noise_ref[...]


def _adj_kernel(cur_ref, up_ref, dn_ref, mask_ref, mn_ref, mx_ref, out_ref):
    i = pl.program_id(0)
    thr = (mx_ref[0, 0] - mn_ref[0, 0]) / float(_M)
    d0 = cur_ref[...]
    up = jnp.concatenate([up_ref[7:8, :], d0[:-1, :]], axis=0)
    dn = jnp.concatenate([d0[1:, :], dn_ref[0:1, :]], axis=0)

    def right(v):
        return jnp.concatenate([v[:, -1:], v[:, :-1]], axis=1)

    def left(v):
        return jnp.concatenate([v[:, 1:], v[:, :1]], axis=1)

    aa = i * _BA + jax.lax.broadcasted_iota(jnp.int32, (_BA, _M), 0)
    bb = jax.lax.broadcasted_iota(jnp.int32, (_BA, _M), 1)
    a_ge1 = aa >= 1
    b_ge1 = bb >= 1
    a_le = aa <= _M - 2
    b_le = bb <= _M - 2

    t1 = (jnp.abs(right(up) - d0) <= thr) & a_ge1 & b_ge1
    t2 = jnp.abs(right(dn) - d0) <= thr
    t3 = (jnp.abs(left(up) - d0) <= thr) & a_ge1 & a_le & b_ge1 & b_le
    t4 = (jnp.abs(left(dn) - d0) <= thr) & a_le & b_le
    adj = t1 | t2 | t3 | t4
    out_ref[...] = adj & (mask_ref[...].astype(jnp.int32) > 0)


@functools.partial(jax.jit)
def kernel(d_coarse):
    m = _M
    noise_t = jax.random.uniform(jax.random.key(42), (m, m), jnp.float32).T
    mask8 = jax.random.bernoulli(jax.random.key(7), 0.5, (m, m)).astype(jnp.int8)

    d2 = d_coarse.reshape(m, 4 * m)
    e, mn, mx = pl.pallas_call(
        _pool_kernel,
        grid=(m // _BM,),
        in_specs=[
            pl.BlockSpec((_BM, 4 * m), lambda i: (i, 0)),
            pl.BlockSpec((m, _BM), lambda i: (0, i)),
        ],
        out_specs=[
            pl.BlockSpec((m, _BM), lambda i: (0, i)),
            pl.BlockSpec((1, 1), lambda i: (0, 0), memory_space=pltpu.SMEM),
            pl.BlockSpec((1, 1), lambda i: (0, 0), memory_space=pltpu.SMEM),
        ],
        out_shape=[
            jax.ShapeDtypeStruct((m, m), jnp.float32),
            jax.ShapeDtypeStruct((1, 1), jnp.float32),
            jax.ShapeDtypeStruct((1, 1), jnp.float32),
        ],
    )(d2, noise_t)

    nb = _BA // 8
    out = pl.pallas_call(
        _adj_kernel,
        grid=(m // _BA,),
        in_specs=[
            pl.BlockSpec((_BA, m), lambda i: (i, 0)),
            pl.BlockSpec((8, m), lambda i: (jnp.maximum(i * nb - 1, 0), 0)),
            pl.BlockSpec((8, m), lambda i: (((i + 1) * nb) % (_M // 8), 0)),
            pl.BlockSpec((_BA, m), lambda i: (i, 0)),
            pl.BlockSpec((1, 1), lambda i: (0, 0), memory_space=pltpu.SMEM),
            pl.BlockSpec((1, 1), lambda i: (0, 0), memory_space=pltpu.SMEM),
        ],
        out_specs=pl.BlockSpec((_BA, m), lambda i: (i, 0)),
        out_shape=jax.ShapeDtypeStruct((m, m), jnp.bool_),
    )(e, e, e, mask8, mn, mx)
    return out


# speedup vs baseline: 28.2964x; 1.0830x over previous
import functools

import jax
import jax.numpy as jnp
from jax.experimental import pallas as pl
from jax.experimental.pallas import tpu as pltpu

_M = 2048
_BM = 128
_BA = 256
_NP = _M // _BM
_NA = _M // _BA
_R0 = 8
_POISON = 1e30


def _fused_kernel(x_ref, noise_ref, pois_ref, out_ref, e_scr, mm_scr):
    g = pl.program_id(0)

    @pl.when(g < _NP)
    def _pool():
        x = x_ref[...]
        y = jnp.maximum(x[:, :4096], x[:, 4096:])
        yt = y.T
        gg = yt.reshape(2048, 2 * _BM)
        pt = jnp.maximum(gg[:, :_BM], gg[:, _BM:])
        bmin = jnp.min(pt)
        bmax = jnp.max(pt)

        @pl.when(g == 0)
        def _init():
            mm_scr[0, 0] = bmin
            mm_scr[1, 0] = bmax

        @pl.when(g > 0)
        def _acc():
            mm_scr[0, 0] = jnp.minimum(mm_scr[0, 0], bmin)
            mm_scr[1, 0] = jnp.maximum(mm_scr[1, 0], bmax)

        eb = pt + noise_ref[...]
        e_scr[pl.ds(_R0, _M), pl.ds(g * _BM, _BM)] = eb
        e_scr[pl.ds(_R0 + _M, 1), pl.ds(g * _BM, _BM)] = eb[0:1, :]

    @pl.when(g >= _NP)
    def _adj():
        i = g - _NP
        a0 = i * _BA
        thr = (mm_scr[1, 0] - mm_scr[0, 0]) / float(_M)
        w = e_scr[pl.ds(a0, _BA + 2 * _R0), :]
        up = w[_R0 - 1:_R0 - 1 + _BA, :]
        cur = w[_R0:_R0 + _BA, :]
        dn = w[_R0 + 1:_R0 + 1 + _BA, :]
        d0 = cur + pois_ref[...].astype(jnp.float32) * _POISON

        rup = jnp.roll(up, 1, axis=1)
        rdn = jnp.roll(dn, 1, axis=1)
        lup = jnp.roll(up, -1, axis=1)
        ldn = jnp.roll(dn, -1, axis=1)

        m1 = jnp.minimum(jnp.abs(rup - d0), jnp.abs(rdn - d0))
        m2 = jnp.minimum(jnp.abs(lup - d0), jnp.abs(ldn - d0))
        adj = jnp.minimum(m1, m2) <= thr
        out_ref[...] = adj

        c0 = (jnp.abs(rdn[:, 0:1] - d0[:, 0:1]) <= thr) | (
            jnp.abs(ldn[:, 0:1] - d0[:, 0:1]) <= thr)
        out_ref[:, 0:1] = c0
        cl = (jnp.abs(rup[:, -1:] - d0[:, -1:]) <= thr) | (
            jnp.abs(rdn[:, -1:] - d0[:, -1:]) <= thr)
        out_ref[:, -1:] = cl

        bb = jax.lax.broadcasted_iota(jnp.int32, (1, _M), 1)

        @pl.when(i == 0)
        def _row0():
            t2 = jnp.abs(rdn[0:1, :] - d0[0:1, :]) <= thr
            t4 = (jnp.abs(ldn[0:1, :] - d0[0:1, :]) <= thr) & (bb <= _M - 2)
            out_ref[0:1, :] = t2 | t4

        @pl.when(i == _NA - 1)
        def _rowl():
            t1 = (jnp.abs(rup[-1:, :] - d0[-1:, :]) <= thr) & (bb >= 1)
            t2 = jnp.abs(rdn[-1:, :] - d0[-1:, :]) <= thr
            out_ref[-1:, :] = t1 | t2


@functools.partial(jax.jit)
def kernel(d_coarse):
    m = _M
    noise_t = jax.random.uniform(jax.random.key(42), (m, m), jnp.float32).T
    mask = jax.random.bernoulli(jax.random.key(7), 0.5, (m, m))
    pois8 = jnp.where(mask, jnp.int8(0), jnp.int8(1))

    d2 = d_coarse.reshape(m, 4 * m)
    out = pl.pallas_call(
        _fused_kernel,
        grid=(_NP + _NA,),
        in_specs=[
            pl.BlockSpec((_BM, 4 * m), lambda g: (jnp.minimum(g, _NP - 1), 0)),
            pl.BlockSpec((m, _BM), lambda g: (0, jnp.minimum(g, _NP - 1))),
            pl.BlockSpec((_BA, m), lambda g: (jnp.maximum(g - _NP, 0), 0)),
        ],
        out_specs=pl.BlockSpec((_BA, m), lambda g: (jnp.maximum(g - _NP, 0), 0)),
        out_shape=jax.ShapeDtypeStruct((m, m), jnp.bool_),
        scratch_shapes=[
            pltpu.VMEM((_R0 + m + 8, m), jnp.float32),
            pltpu.SMEM((2, 1), jnp.float32),
        ],
    )(d2, noise_t, pois8)
    return out
